# baseline (speedup 1.0000x reference)
import jax
import jax.numpy as jnp
from jax.experimental import pallas as pl
from jax.experimental.pallas import tpu as pltpu

_B, _H, _W = 32, 512, 512
_RC = 4


def _loss_kernel(lab_ref, xw_ref, xs_ref, pm_ref, nm_ref, num_ref, cnt_ref):
    b = pl.program_id(0)
    c = pl.program_id(1)

    @pl.when((b == 0) & (c == 0))
    def _init():
        num_ref[:, :] = jnp.zeros((1, 1), jnp.float32)
        cnt_ref[:, :] = jnp.zeros((1, 1), jnp.float32)

    xw = xw_ref[0]
    xs = xs_ref[0]
    mask = (pm_ref[0] > 0.5) & (nm_ref[0] > 0.5)
    y = lab_ref[b].astype(jnp.float32)

    sp_w = jnp.maximum(-xw, 0.0) + jnp.log1p(jnp.exp(-jnp.abs(xw)))
    sp_s = jnp.maximum(-xs, 0.0) + jnp.log1p(jnp.exp(-jnp.abs(xs)))

    pseudo = jnp.where((xw > 0.6) | (xw < 0.4), xw, 0.0)
    t_sum = (sp_w + sp_s) + (xw + xs) - y * xw - pseudo * xs

    num_ref[:, :] += jnp.sum(jnp.where(mask, t_sum, 0.0)).reshape(1, 1)
    cnt_ref[:, :] += jnp.sum(jnp.where(mask, 1.0, 0.0)).reshape(1, 1)


def kernel(logits_w, logits_s, prostate_mask, needle_mask, ood_mask,
           label, involvement):
    del ood_mask, involvement
    xw = logits_w.reshape(_B, _H, _W)
    xs = logits_s.reshape(_B, _H, _W)
    pm = prostate_mask.reshape(_B, _H, _W)
    nm = needle_mask.reshape(_B, _H, _W)

    rows = _H // _RC
    blk = pl.BlockSpec((1, rows, _W), lambda b, c, lab: (b, c, 0))
    out_blk = pl.BlockSpec((1, 1), lambda b, c, lab: (0, 0))
    scal = jax.ShapeDtypeStruct((1, 1), jnp.float32)

    num, cnt = pl.pallas_call(
        _loss_kernel,
        grid_spec=pltpu.PrefetchScalarGridSpec(
            num_scalar_prefetch=1,
            grid=(_B, _RC),
            in_specs=[blk, blk, blk, blk],
            out_specs=[out_blk, out_blk],
        ),
        out_shape=[scal, scal],
    )(label.astype(jnp.int32), xw, xs, pm, nm)

    return (0.5 * num[0, 0] / cnt[0, 0]).astype(jnp.float32)


# speedup vs baseline: 1.3631x; 1.3631x over previous
import jax
import jax.numpy as jnp
from jax.experimental import pallas as pl
from jax.experimental.pallas import tpu as pltpu

_B, _H, _W = 32, 512, 512
_RC = 1


def _loss_kernel(lab_ref, xw_ref, xs_ref, pm_ref, nm_ref, num_ref, cnt_ref):
    b = pl.program_id(0)
    c = pl.program_id(1)

    @pl.when((b == 0) & (c == 0))
    def _init():
        num_ref[:, :] = jnp.zeros((1, 1), jnp.float32)
        cnt_ref[:, :] = jnp.zeros((1, 1), jnp.float32)

    xw = xw_ref[0]
    xs = xs_ref[0]
    mask = (pm_ref[0] > 0.5) & (nm_ref[0] > 0.5)
    y = lab_ref[b].astype(jnp.float32)

    sp_w = jnp.maximum(-xw, 0.0) + jnp.log1p(jnp.exp(-jnp.abs(xw)))
    sp_s = jnp.maximum(-xs, 0.0) + jnp.log1p(jnp.exp(-jnp.abs(xs)))

    pseudo = jnp.where((xw > 0.6) | (xw < 0.4), xw, 0.0)
    t_sum = (sp_w + sp_s) + (xw + xs) - y * xw - pseudo * xs

    num_ref[:, :] += jnp.sum(jnp.where(mask, t_sum, 0.0)).reshape(1, 1)
    cnt_ref[:, :] += jnp.sum(jnp.where(mask, 1.0, 0.0)).reshape(1, 1)


def kernel(logits_w, logits_s, prostate_mask, needle_mask, ood_mask,
           label, involvement):
    del ood_mask, involvement
    xw = logits_w.reshape(_B, _H, _W)
    xs = logits_s.reshape(_B, _H, _W)
    pm = prostate_mask.reshape(_B, _H, _W)
    nm = needle_mask.reshape(_B, _H, _W)

    rows = _H // _RC
    blk = pl.BlockSpec((1, rows, _W), lambda b, c, lab: (b, c, 0))
    out_blk = pl.BlockSpec((1, 1), lambda b, c, lab: (0, 0))
    scal = jax.ShapeDtypeStruct((1, 1), jnp.float32)

    num, cnt = pl.pallas_call(
        _loss_kernel,
        grid_spec=pltpu.PrefetchScalarGridSpec(
            num_scalar_prefetch=1,
            grid=(_B, _RC),
            in_specs=[blk, blk, blk, blk],
            out_specs=[out_blk, out_blk],
        ),
        out_shape=[scal, scal],
    )(label.astype(jnp.int32), xw, xs, pm, nm)

    return (0.5 * num[0, 0] / cnt[0, 0]).astype(jnp.float32)
